# Initial kernel scaffold; baseline (speedup 1.0000x reference)
import jax
import jax.numpy as jnp
from jax.experimental import pallas as pl


def kernel(user, item, user_embedding, item_embedding, user_bias, item_bias, global_bias):
    raise NotImplementedError("write your pallas kernel here")


# speedup vs baseline: 1.5888x; 1.5888x over previous
import functools

import jax
import jax.numpy as jnp
from jax import lax
from jax.experimental import pallas as pl
from jax.experimental.pallas import tpu as pltpu
from jax.experimental.pallas import tpu_sc as plsc

NC = 2
NS = 16
L = 16
NW = NC * NS

BATCH = 16384
D = 128
BPW = BATCH // NW
C = 128
NCHUNK = BPW // C
SEG = D // L
GROUPS = C // L


def _mf_body(uidx_hbm, iidx_hbm, uemb_hbm, iemb_hbm, ub_hbm, ib_hbm, gb_hbm,
             out_hbm,
             uidx_v, iidx_v, u_buf, i_buf, ub_buf, ib_buf, out_v, tp_v, gb_v,
             sem0, sem1):
  wid = lax.axis_index("s") * NC + lax.axis_index("c")
  base = wid * BPW

  pltpu.sync_copy(uidx_hbm.at[wid], uidx_v)
  pltpu.sync_copy(iidx_hbm.at[wid], iidx_v)
  pltpu.sync_copy(gb_hbm, gb_v)

  sems = (sem0, sem1)
  handles = [None, None]

  def fire(j):
    slot = j % 2
    sem = sems[slot]
    uix = uidx_v.at[j]
    iix = iidx_v.at[j]
    handles[slot] = (
        pltpu.async_copy(uemb_hbm.at[uix], u_buf.at[slot], sem),
        pltpu.async_copy(iemb_hbm.at[iix], i_buf.at[slot], sem),
        pltpu.async_copy(ub_hbm.at[uix], ub_buf.at[slot], sem),
        pltpu.async_copy(ib_hbm.at[iix], ib_buf.at[slot], sem),
    )

  kidx = lax.iota(jnp.int32, L) * L
  gbias = gb_v[pl.ds(0, L)]

  fire(0)
  for j in range(NCHUNK):
    if j + 1 < NCHUNK:
      fire(j + 1)
    slot = j % 2
    for h in handles[slot]:
      h.wait()

    def group_body(g, _, slot=slot, j=j):
      rowbase = g * L
      for r in range(L):
        row = rowbase + r
        acc = (u_buf[slot, row, pl.ds(0, L)] *
               i_buf[slot, row, pl.ds(0, L)])
        for s in range(1, SEG):
          acc = acc + (u_buf[slot, row, pl.ds(s * L, L)] *
                       i_buf[slot, row, pl.ds(s * L, L)])
        plsc.store_scatter(tp_v, [kidx + r], acc)
      red = tp_v[pl.ds(0, L)]
      for k in range(1, L):
        red = red + tp_v[pl.ds(k * L, L)]
      ubv = ub_buf[slot, pl.ds(rowbase, L)]
      ibv = ib_buf[slot, pl.ds(rowbase, L)]
      out_v[pl.ds(j * C + rowbase, L)] = red + ubv + ibv + gbias
      return 0

    lax.fori_loop(0, GROUPS, group_body, 0)

  pltpu.sync_copy(out_v, out_hbm.at[pl.ds(base, BPW)])


@jax.jit
def kernel(user, item, user_embedding, item_embedding, user_bias, item_bias,
           global_bias):
  uidx = user.reshape(NW, NCHUNK, C)
  iidx = item.reshape(NW, NCHUNK, C)
  ub = user_bias.reshape(-1)
  ib = item_bias.reshape(-1)
  gb = jnp.broadcast_to(global_bias, (L,))

  mesh = plsc.VectorSubcoreMesh(core_axis_name="c", subcore_axis_name="s",
                                num_cores=NC, num_subcores=NS)
  run = pl.kernel(
      _mf_body,
      out_type=jax.ShapeDtypeStruct((BATCH,), jnp.float32),
      mesh=mesh,
      compiler_params=pltpu.CompilerParams(needs_layout_passes=False),
      scratch_types=[
          pltpu.VMEM((NCHUNK, C), jnp.int32),
          pltpu.VMEM((NCHUNK, C), jnp.int32),
          pltpu.VMEM((2, C, D), jnp.float32),
          pltpu.VMEM((2, C, D), jnp.float32),
          pltpu.VMEM((2, C), jnp.float32),
          pltpu.VMEM((2, C), jnp.float32),
          pltpu.VMEM((BPW,), jnp.float32),
          pltpu.VMEM((L * L,), jnp.float32),
          pltpu.VMEM((L,), jnp.float32),
          pltpu.SemaphoreType.DMA,
          pltpu.SemaphoreType.DMA,
      ],
  )
  return run(uidx, iidx, user_embedding, item_embedding, ub, ib, gb)


# speedup vs baseline: 1.6942x; 1.0664x over previous
import functools

import jax
import jax.numpy as jnp
from jax import lax
from jax.experimental import pallas as pl
from jax.experimental.pallas import tpu as pltpu
from jax.experimental.pallas import tpu_sc as plsc

NC = 2
NS = 16
L = 16
NW = NC * NS

BATCH = 16384
D = 128
BPW = BATCH // NW
C = 128
NCHUNK = BPW // C
SEG = D // L
GROUPS = C // L


def _mf_body(uidx_hbm, iidx_hbm, uemb_hbm, iemb_hbm, ub_hbm, ib_hbm, gb_hbm,
             out_hbm,
             uidx_v, iidx_v, u_buf, i_buf, ub_buf, ib_buf, out_v, tp_v, gb_v,
             sem0, sem1):
  wid = lax.axis_index("s") * NC + lax.axis_index("c")
  base = wid * BPW

  pltpu.sync_copy(uidx_hbm.at[pl.ds(base, BPW)], uidx_v)
  pltpu.sync_copy(iidx_hbm.at[pl.ds(base, BPW)], iidx_v)
  pltpu.sync_copy(gb_hbm, gb_v.at[pl.ds(0, 1)])

  sems = (sem0, sem1)
  handles = [None, None]

  def fire(j):
    slot = j % 2
    sem = sems[slot]
    uix = uidx_v.at[pl.ds(j * C, C)]
    iix = iidx_v.at[pl.ds(j * C, C)]
    handles[slot] = (
        pltpu.async_copy(uemb_hbm.at[uix], u_buf.at[slot], sem),
        pltpu.async_copy(iemb_hbm.at[iix], i_buf.at[slot], sem),
        pltpu.async_copy(ub_hbm.at[uix], ub_buf.at[slot], sem),
        pltpu.async_copy(ib_hbm.at[iix], ib_buf.at[slot], sem),
    )

  kidx = lax.iota(jnp.int32, L) * L
  zidx = jnp.zeros((L,), jnp.int32)
  gbias = plsc.load_gather(gb_v, [zidx])

  fire(0)
  for j in range(NCHUNK):
    if j + 1 < NCHUNK:
      fire(j + 1)
    slot = j % 2
    for h in handles[slot]:
      h.wait()

    def group_body(g, _, slot=slot, j=j):
      rowbase = g * L
      for r in range(L):
        row = rowbase + r
        acc = (u_buf[slot, row, pl.ds(0, L)] *
               i_buf[slot, row, pl.ds(0, L)])
        for s in range(1, SEG):
          acc = acc + (u_buf[slot, row, pl.ds(s * L, L)] *
                       i_buf[slot, row, pl.ds(s * L, L)])
        plsc.store_scatter(tp_v, [kidx + r], acc)
      red = tp_v[pl.ds(0, L)]
      for k in range(1, L):
        red = red + tp_v[pl.ds(k * L, L)]
      ubv = ub_buf[slot, pl.ds(rowbase, L)]
      ibv = ib_buf[slot, pl.ds(rowbase, L)]
      out_v[pl.ds(j * C + rowbase, L)] = red + ubv + ibv + gbias
      return 0

    lax.fori_loop(0, GROUPS, group_body, 0)

  pltpu.sync_copy(out_v, out_hbm.at[pl.ds(base, BPW)])


@jax.jit
def kernel(user, item, user_embedding, item_embedding, user_bias, item_bias,
           global_bias):
  ub = user_bias.reshape(-1)
  ib = item_bias.reshape(-1)

  mesh = plsc.VectorSubcoreMesh(core_axis_name="c", subcore_axis_name="s",
                                num_cores=NC, num_subcores=NS)
  run = pl.kernel(
      _mf_body,
      out_type=jax.ShapeDtypeStruct((BATCH,), jnp.float32),
      mesh=mesh,
      compiler_params=pltpu.CompilerParams(needs_layout_passes=False),
      scratch_types=[
          pltpu.VMEM((BPW,), jnp.int32),
          pltpu.VMEM((BPW,), jnp.int32),
          pltpu.VMEM((2, C, D), jnp.float32),
          pltpu.VMEM((2, C, D), jnp.float32),
          pltpu.VMEM((2, C), jnp.float32),
          pltpu.VMEM((2, C), jnp.float32),
          pltpu.VMEM((BPW,), jnp.float32),
          pltpu.VMEM((L * L,), jnp.float32),
          pltpu.VMEM((L,), jnp.float32),
          pltpu.SemaphoreType.DMA,
          pltpu.SemaphoreType.DMA,
      ],
  )
  return run(user, item, user_embedding, item_embedding, ub, ib, global_bias)


# speedup vs baseline: 1.7214x; 1.0160x over previous
import functools

import jax
import jax.numpy as jnp
from jax import lax
from jax.experimental import pallas as pl
from jax.experimental.pallas import tpu as pltpu
from jax.experimental.pallas import tpu_sc as plsc

NC = 2
NS = 16
L = 16
NW = NC * NS

BATCH = 16384
D = 128
BPW = BATCH // NW
C = 128
NCHUNK = BPW // C
SEG = D // L
GROUPS = C // L


def _mf_body(uidx_hbm, iidx_hbm, uemb_hbm, iemb_hbm, ub_hbm, ib_hbm, gb_hbm,
             out_hbm,
             uidx_v, iidx_v, u_buf, i_buf, ub_buf, ib_buf, out_v, tp_v, gb_v,
             sem0, sem1, semb):
  wid = lax.axis_index("s") * NC + lax.axis_index("c")
  base = wid * BPW

  pltpu.sync_copy(uidx_hbm.at[pl.ds(base, BPW)], uidx_v)
  pltpu.sync_copy(iidx_hbm.at[pl.ds(base, BPW)], iidx_v)
  pltpu.sync_copy(gb_hbm, gb_v.at[pl.ds(0, 1)])

  sems = (sem0, sem1)

  def emb_copies(j, slot):
    uix = uidx_v.at[pl.ds(j * C, C)]
    iix = iidx_v.at[pl.ds(j * C, C)]
    return (
        pltpu.make_async_copy(uemb_hbm.at[uix], u_buf.at[slot], sems[slot]),
        pltpu.make_async_copy(iemb_hbm.at[iix], i_buf.at[slot], sems[slot]),
    )

  bias_handles = []
  for j in range(NCHUNK):
    uix = uidx_v.at[pl.ds(j * C, C)]
    iix = iidx_v.at[pl.ds(j * C, C)]
    bias_handles.append(
        pltpu.async_copy(ub_hbm.at[uix], ub_buf.at[pl.ds(j * C, C)], semb))
    bias_handles.append(
        pltpu.async_copy(ib_hbm.at[iix], ib_buf.at[pl.ds(j * C, C)], semb))
  for slot in range(2):
    for c in emb_copies(slot, slot):
      c.start()
  for h in bias_handles:
    h.wait()

  kidx = lax.iota(jnp.int32, L) * L
  zidx = jnp.zeros((L,), jnp.int32)
  gbias = plsc.load_gather(gb_v, [zidx])

  def outer_body(it, _):
    for slot in range(2):
      j = it * 2 + slot
      for c in emb_copies(j, slot):
        c.wait()

      def group_body(g, _, slot=slot):
        rowbase = g * L
        for r in range(L):
          row = rowbase + r
          acc = (u_buf[slot, row, pl.ds(0, L)] *
                 i_buf[slot, row, pl.ds(0, L)])
          for s in range(1, SEG):
            acc = acc + (u_buf[slot, row, pl.ds(s * L, L)] *
                         i_buf[slot, row, pl.ds(s * L, L)])
          plsc.store_scatter(tp_v, [kidx + r], acc)
        red = tp_v[pl.ds(0, L)]
        for k in range(1, L):
          red = red + tp_v[pl.ds(k * L, L)]
        pos = j * C + rowbase
        ubv = ub_buf[pl.ds(pos, L)]
        ibv = ib_buf[pl.ds(pos, L)]
        out_v[pl.ds(pos, L)] = red + ubv + ibv + gbias
        return 0

      lax.fori_loop(0, GROUPS, group_body, 0)

      @pl.when(it == 0)
      def _fire_next(slot=slot, j=j):
        for c in emb_copies(j + 2, slot):
          c.start()
    return 0

  lax.fori_loop(0, NCHUNK // 2, outer_body, 0)

  pltpu.sync_copy(out_v, out_hbm.at[pl.ds(base, BPW)])


@jax.jit
def kernel(user, item, user_embedding, item_embedding, user_bias, item_bias,
           global_bias):
  ub = user_bias.reshape(-1)
  ib = item_bias.reshape(-1)

  mesh = plsc.VectorSubcoreMesh(core_axis_name="c", subcore_axis_name="s",
                                num_cores=NC, num_subcores=NS)
  run = pl.kernel(
      _mf_body,
      out_type=jax.ShapeDtypeStruct((BATCH,), jnp.float32),
      mesh=mesh,
      compiler_params=pltpu.CompilerParams(needs_layout_passes=False),
      scratch_types=[
          pltpu.VMEM((BPW,), jnp.int32),
          pltpu.VMEM((BPW,), jnp.int32),
          pltpu.VMEM((2, C, D), jnp.float32),
          pltpu.VMEM((2, C, D), jnp.float32),
          pltpu.VMEM((BPW,), jnp.float32),
          pltpu.VMEM((BPW,), jnp.float32),
          pltpu.VMEM((BPW,), jnp.float32),
          pltpu.VMEM((L * L,), jnp.float32),
          pltpu.VMEM((L,), jnp.float32),
          pltpu.SemaphoreType.DMA,
          pltpu.SemaphoreType.DMA,
          pltpu.SemaphoreType.DMA,
      ],
  )
  return run(user, item, user_embedding, item_embedding, ub, ib, global_bias)


# speedup vs baseline: 1.7304x; 1.0052x over previous
import functools

import jax
import jax.numpy as jnp
from jax import lax
from jax.experimental import pallas as pl
from jax.experimental.pallas import tpu as pltpu
from jax.experimental.pallas import tpu_sc as plsc

NC = 2
NS = 16
L = 16
NW = NC * NS

BATCH = 16384
D = 128
BPW = BATCH // NW
C = 128
NCHUNK = BPW // C
SEG = D // L
GROUPS = C // L


def _mf_body(uidx_hbm, iidx_hbm, uemb_hbm, iemb_hbm, ub_hbm, ib_hbm, gb_hbm,
             out_hbm,
             uidx_v, iidx_v, u_buf, i_buf, ub_buf, ib_buf, out_v, tp_v, gb_v,
             sem0, sem1, semb):
  wid = lax.axis_index("s") * NC + lax.axis_index("c")
  base = wid * BPW

  pltpu.sync_copy(uidx_hbm.at[pl.ds(base, BPW)], uidx_v)
  pltpu.sync_copy(iidx_hbm.at[pl.ds(base, BPW)], iidx_v)
  pltpu.sync_copy(gb_hbm, gb_v.at[pl.ds(0, 1)])

  sems = (sem0, sem1)
  handles = [None, None]

  def fire(j):
    slot = j % 2
    sem = sems[slot]
    uix = uidx_v.at[pl.ds(j * C, C)]
    iix = iidx_v.at[pl.ds(j * C, C)]
    handles[slot] = (
        pltpu.async_copy(uemb_hbm.at[uix], u_buf.at[slot], sem),
        pltpu.async_copy(iemb_hbm.at[iix], i_buf.at[slot], sem),
    )

  fire(0)
  bias_handles = []
  for j in range(NCHUNK):
    uix = uidx_v.at[pl.ds(j * C, C)]
    iix = iidx_v.at[pl.ds(j * C, C)]
    bias_handles.append(
        pltpu.async_copy(ub_hbm.at[uix], ub_buf.at[pl.ds(j * C, C)], semb))
    bias_handles.append(
        pltpu.async_copy(ib_hbm.at[iix], ib_buf.at[pl.ds(j * C, C)], semb))
  for h in bias_handles:
    h.wait()

  kidx = lax.iota(jnp.int32, L) * L
  zidx = jnp.zeros((L,), jnp.int32)
  gbias = plsc.load_gather(gb_v, [zidx])

  for j in range(NCHUNK):
    if j + 1 < NCHUNK:
      fire(j + 1)
    slot = j % 2
    for h in handles[slot]:
      h.wait()

    def group_body(g, _, slot=slot, j=j):
      rowbase = g * L
      for r in range(L):
        row = rowbase + r
        acc = (u_buf[slot, row, pl.ds(0, L)] *
               i_buf[slot, row, pl.ds(0, L)])
        for s in range(1, SEG):
          acc = acc + (u_buf[slot, row, pl.ds(s * L, L)] *
                       i_buf[slot, row, pl.ds(s * L, L)])
        plsc.store_scatter(tp_v, [kidx + r], acc)
      red = tp_v[pl.ds(0, L)]
      for k in range(1, L):
        red = red + tp_v[pl.ds(k * L, L)]
      pos = j * C + rowbase
      ubv = ub_buf[pl.ds(pos, L)]
      ibv = ib_buf[pl.ds(pos, L)]
      out_v[pl.ds(pos, L)] = red + ubv + ibv + gbias
      return 0

    lax.fori_loop(0, GROUPS, group_body, 0)

  pltpu.sync_copy(out_v, out_hbm.at[pl.ds(base, BPW)])


@jax.jit
def kernel(user, item, user_embedding, item_embedding, user_bias, item_bias,
           global_bias):
  ub = user_bias.reshape(-1)
  ib = item_bias.reshape(-1)

  mesh = plsc.VectorSubcoreMesh(core_axis_name="c", subcore_axis_name="s",
                                num_cores=NC, num_subcores=NS)
  run = pl.kernel(
      _mf_body,
      out_type=jax.ShapeDtypeStruct((BATCH,), jnp.float32),
      mesh=mesh,
      compiler_params=pltpu.CompilerParams(needs_layout_passes=False),
      scratch_types=[
          pltpu.VMEM((BPW,), jnp.int32),
          pltpu.VMEM((BPW,), jnp.int32),
          pltpu.VMEM((2, C, D), jnp.float32),
          pltpu.VMEM((2, C, D), jnp.float32),
          pltpu.VMEM((BPW,), jnp.float32),
          pltpu.VMEM((BPW,), jnp.float32),
          pltpu.VMEM((BPW,), jnp.float32),
          pltpu.VMEM((L * L,), jnp.float32),
          pltpu.VMEM((L,), jnp.float32),
          pltpu.SemaphoreType.DMA,
          pltpu.SemaphoreType.DMA,
          pltpu.SemaphoreType.DMA,
      ],
  )
  return run(user, item, user_embedding, item_embedding, ub, ib, global_bias)
